# baseline (speedup 1.0000x reference)
import jax
import jax.numpy as jnp
from jax import lax
from jax.experimental import pallas as pl
from jax.experimental.pallas import tpu as pltpu
from jax.experimental.pallas import tpu_sc as plsc

_B = 65536
_SIN = 64
_SOUT = 256
_L = 16
_NC = 2
_NS = 16
_NW = _NC * _NS
_ROWS_PER_W = _B // _NW
_CHUNK = 128
_NCHUNK = _ROWS_PER_W // _CHUNK
_UNROLL = 4
_PAD = _SIN + 1


def _sc_body(xt_hbm, w_hbm, no_hbm, out_hbm, no_v, w_v, xbuf, xpad, obuf,
             isem0, isem1, osem0, osem1):
    wid = lax.axis_index("s") * _NC + lax.axis_index("c")
    base = wid * _ROWS_PER_W

    pltpu.sync_copy(no_hbm, no_v)
    pltpu.sync_copy(w_hbm, w_v)
    nov = [no_v[pl.ds(k * _L, _L)] for k in range(_SIN // _L)]
    wv = [w_v[pl.ds(k * _L, _L)] for k in range(_SIN // _L)]
    rvec = [lax.iota(jnp.int32, _L) + j * _L for j in range(_CHUNK // _L)]

    zf = jnp.zeros((_L,), jnp.float32)

    def zero_body(r, c):
        for b in range(2):
            for j in range(_SOUT // _L):
                obuf[b, r, pl.ds(j * _L, _L)] = zf
        return c

    lax.fori_loop(0, _CHUNK, zero_body, 0)

    isems = [isem0, isem1]
    osems = [osem0, osem1]

    for b in range(2):
        pltpu.async_copy(
            xt_hbm.at[:, pl.ds(base + b * _CHUNK, _CHUNK)], xbuf.at[b], isems[b]
        )

    def outer(t, carry):
        for b in range(2):
            chunk = 2 * t + b
            r0 = base + chunk * _CHUNK
            pltpu.make_async_copy(
                xt_hbm.at[:, pl.ds(r0, _CHUNK)], xbuf.at[b], isems[b]
            ).wait()

            def tr_body(g, cc):
                i = g * _UNROLL
                isps = [
                    jnp.full((_L,), i + u, jnp.int32) for u in range(_UNROLL)
                ]
                for j in range(_CHUNK // _L):
                    for u in range(_UNROLL):
                        v = xbuf[b, i + u, pl.ds(j * _L, _L)]
                        plsc.store_scatter(xpad.at[b], [rvec[j], isps[u]], v)
                return cc

            lax.fori_loop(0, _SIN // _UNROLL, tr_body, 0)

            @pl.when(chunk + 2 < _NCHUNK)
            def _next_in():
                pltpu.async_copy(
                    xt_hbm.at[:, pl.ds(r0 + 2 * _CHUNK, _CHUNK)], xbuf.at[b], isems[b]
                )

            @pl.when(t > 0)
            def _wait_out():
                pltpu.make_async_copy(
                    obuf.at[b], out_hbm.at[pl.ds(r0, _CHUNK)], osems[b]
                ).wait()

            def row_body(i, cc):
                r = i * _UNROLL
                for u in range(_UNROLL):
                    rs = jnp.full((_L,), r + u, jnp.int32)
                    for k in range(_SIN // _L):
                        g = xpad[b, r + u, pl.ds(k * _L, _L)] * wv[k]
                        plsc.store_scatter(obuf.at[b], [rs, nov[k]], g)
                return cc

            lax.fori_loop(0, _CHUNK // _UNROLL, row_body, 0)

            pltpu.async_copy(obuf.at[b], out_hbm.at[pl.ds(r0, _CHUNK)], osems[b])

        return carry

    lax.fori_loop(0, _NCHUNK // 2, outer, 0)

    for b in range(2):
        pltpu.make_async_copy(
            obuf.at[b], out_hbm.at[pl.ds(base, _CHUNK)], osems[b]
        ).wait()


def _make_call():
    return pl.kernel(
        _sc_body,
        name="scatter_cols",
        out_type=jax.ShapeDtypeStruct((_B, _SOUT), jnp.float32),
        mesh=plsc.VectorSubcoreMesh(
            core_axis_name="c", subcore_axis_name="s", num_cores=_NC, num_subcores=_NS
        ),
        compiler_params=pltpu.CompilerParams(needs_layout_passes=False),
        scratch_types=[
            pltpu.VMEM((_SIN,), jnp.int32),
            pltpu.VMEM((_SIN,), jnp.float32),
            pltpu.VMEM((2, _SIN, _CHUNK), jnp.float32),
            pltpu.VMEM((2, _CHUNK, _PAD), jnp.float32),
            pltpu.VMEM((2, _CHUNK, _SOUT), jnp.float32),
            pltpu.SemaphoreType.DMA,
            pltpu.SemaphoreType.DMA,
            pltpu.SemaphoreType.DMA,
            pltpu.SemaphoreType.DMA,
        ],
    )


@jax.jit
def kernel(x, weights, node_order):
    return _make_call()(x.T, weights, node_order)


# speedup vs baseline: 1.4184x; 1.4184x over previous
import functools

import jax
import jax.numpy as jnp
from jax import lax
from jax.experimental import pallas as pl
from jax.experimental.pallas import tpu as pltpu
from jax.experimental.pallas import tpu_sc as plsc

_B = 65536
_SIN = 64
_SOUT = 256
_L = 16
_NC = 2
_NS = 16
_NW = _NC * _NS
_NSPLIT = 2
_BSPLIT = _B // _NSPLIT
_ROWS_PER_W = _BSPLIT // _NW
_CHUNK = 128
_NCHUNK = _ROWS_PER_W // _CHUNK
_UNROLL = 4


def _sc_body(phase, x_hbm, w_hbm, no_hbm, out_hbm, no_v, w_v, xbuf, obuf,
             isem0, isem1, osem0, osem1):
    wid = lax.axis_index("s") * _NC + lax.axis_index("c")
    base = wid * _ROWS_PER_W
    obase = phase * _BSPLIT + base

    pltpu.sync_copy(no_hbm, no_v)
    pltpu.sync_copy(w_hbm, w_v)
    nov = [no_v[pl.ds(k * _L, _L)] for k in range(_SIN // _L)]
    wv = [w_v[pl.ds(k * _L, _L)] for k in range(_SIN // _L)]

    zf = jnp.zeros((_L,), jnp.float32)

    def zero_body(r, c):
        for b in range(2):
            for j in range(_SOUT // _L):
                obuf[b, r, pl.ds(j * _L, _L)] = zf
        return c

    lax.fori_loop(0, _CHUNK, zero_body, 0)

    isems = [isem0, isem1]
    osems = [osem0, osem1]

    for b in range(2):
        pltpu.async_copy(
            x_hbm.at[pl.ds(base + b * _CHUNK, _CHUNK)], xbuf.at[b], isems[b]
        )

    def outer(t, carry):
        for b in range(2):
            chunk = 2 * t + b
            r0 = base + chunk * _CHUNK
            o0 = obase + chunk * _CHUNK
            pltpu.make_async_copy(
                x_hbm.at[pl.ds(r0, _CHUNK)], xbuf.at[b], isems[b]
            ).wait()

            @pl.when(t > 0)
            def _wait_out():
                pltpu.make_async_copy(
                    obuf.at[b], out_hbm.at[pl.ds(o0, _CHUNK)], osems[b]
                ).wait()

            def row_body(i, cc):
                r = i * _UNROLL
                for u in range(_UNROLL):
                    rs = jnp.full((_L,), r + u, jnp.int32)
                    for k in range(_SIN // _L):
                        v = xbuf[b, r + u, pl.ds(k * _L, _L)] * wv[k]
                        plsc.store_scatter(obuf.at[b], [rs, nov[k]], v)
                return cc

            lax.fori_loop(0, _CHUNK // _UNROLL, row_body, 0)

            pltpu.async_copy(obuf.at[b], out_hbm.at[pl.ds(o0, _CHUNK)], osems[b])

            @pl.when(chunk + 2 < _NCHUNK)
            def _next_in():
                pltpu.async_copy(
                    x_hbm.at[pl.ds(r0 + 2 * _CHUNK, _CHUNK)], xbuf.at[b], isems[b]
                )

        return carry

    lax.fori_loop(0, _NCHUNK // 2, outer, 0)

    for b in range(2):
        pltpu.make_async_copy(
            obuf.at[b], out_hbm.at[pl.ds(obase, _CHUNK)], osems[b]
        ).wait()


def _make_call(phase):
    return pl.kernel(
        functools.partial(_sc_body, phase),
        name=f"scatter_cols_p{phase}",
        out_type=(
            jax.ShapeDtypeStruct((_B, _SOUT), jnp.float32) if phase == 0 else ()
        ),
        mesh=plsc.VectorSubcoreMesh(
            core_axis_name="c", subcore_axis_name="s", num_cores=_NC, num_subcores=_NS
        ),
        compiler_params=pltpu.CompilerParams(needs_layout_passes=False),
        scratch_types=[
            pltpu.VMEM((_SIN,), jnp.int32),
            pltpu.VMEM((_SIN,), jnp.float32),
            pltpu.VMEM((2, _CHUNK, _SIN), jnp.float32),
            pltpu.VMEM((2, _CHUNK, _SOUT), jnp.float32),
            pltpu.SemaphoreType.DMA,
            pltpu.SemaphoreType.DMA,
            pltpu.SemaphoreType.DMA,
            pltpu.SemaphoreType.DMA,
        ],
    )


@jax.jit
def kernel(x, weights, node_order):
    xs = [
        lax.slice(x, (s * _BSPLIT, 0), ((s + 1) * _BSPLIT, _SIN))
        for s in range(_NSPLIT)
    ]
    out = _make_call(0)(xs[0], weights, node_order)
    out_ref = jax.new_ref(out)
    for s in range(1, _NSPLIT):
        _make_call(s)(xs[s], weights, node_order, out_ref)
    return out_ref[...]
